# baseline (speedup 1.0000x reference)
import functools

import jax
import jax.numpy as jnp
from jax import lax
from jax.experimental import pallas as pl
from jax.experimental.pallas import tpu as pltpu
from jax.experimental.pallas import tpu_sc as plsc

_E = 5
_K = 3
_NW = 32
_CHUNK = 128


def _gating_logits_kernel(x_ref, wg_ref, bg_ref, o_ref):
    logits = jnp.dot(x_ref[...], wg_ref[...],
                     preferred_element_type=jnp.float32) + bg_ref[...]
    o_ref[...] = logits.T


def _sc_mask_kernel(lt_hbm, out_hbm, lchunk, pchunk, *, n_tokens):
    wid = lax.axis_index("s") * 2 + lax.axis_index("c")
    base = wid * _CHUNK
    for e in range(_E):
        pltpu.sync_copy(lt_hbm.at[pl.ds(e * n_tokens + base, _CHUNK)],
                        lchunk.at[pl.ds(e * _CHUNK, _CHUNK)])
    for i in range(_CHUNK // 16):
        ls = [lchunk[pl.ds(e * _CHUNK + i * 16, 16)] for e in range(_E)]
        m = ls[0]
        for e in range(1, _E):
            m = jnp.maximum(m, ls[e])
        es = [jnp.exp(l - m) for l in ls]
        s = es[0] + es[1] + es[2] + es[3] + es[4]
        gs = [v / s for v in es]
        for e in range(_E):
            cnt = jnp.zeros((16,), jnp.float32)
            for e2 in range(_E):
                if e2 == e:
                    continue
                if e2 < e:
                    beats = gs[e2] >= gs[e]
                else:
                    beats = gs[e2] > gs[e]
                cnt = cnt + jnp.where(beats, 1.0, 0.0)
            pchunk[pl.ds(e * _CHUNK + i * 16, 16)] = jnp.where(
                cnt < float(_K), gs[e], 0.0)
    for e in range(_E):
        pltpu.sync_copy(pchunk.at[pl.ds(e * _CHUNK, _CHUNK)],
                        out_hbm.at[pl.ds(e * n_tokens + base, _CHUNK)])


def _combine_kernel(p_ref, x_ref, we_ref, be_ref, o_ref):
    x = x_ref[...]
    p = p_ref[...]
    acc = jnp.zeros((x.shape[0], o_ref.shape[1]), jnp.float32)
    for e in range(_E):
        y_e = jnp.dot(x, we_ref[e], preferred_element_type=jnp.float32)
        acc = acc + p[:, e:e + 1] * (y_e + be_ref[e][None, :])
    o_ref[...] = acc


def kernel(x, Wg, bg, We, be):
    B, D = x.shape
    E, _, H = We.shape
    bg2 = bg.reshape(1, E)

    Tg = 512
    lt = pl.pallas_call(
        _gating_logits_kernel,
        grid=(B // Tg,),
        in_specs=[
            pl.BlockSpec((Tg, D), lambda t: (t, 0)),
            pl.BlockSpec((D, E), lambda t: (0, 0)),
            pl.BlockSpec((1, E), lambda t: (0, 0)),
        ],
        out_specs=pl.BlockSpec((E, Tg), lambda t: (0, t)),
        out_shape=jax.ShapeDtypeStruct((E, B), jnp.float32),
    )(x, Wg, bg2)

    mesh = plsc.VectorSubcoreMesh(core_axis_name="c", subcore_axis_name="s")
    sc_mask = functools.partial(
        pl.kernel,
        mesh=mesh,
        out_type=jax.ShapeDtypeStruct((E * B,), jnp.float32),
        scratch_types=[
            pltpu.VMEM((_E * _CHUNK,), jnp.float32),
            pltpu.VMEM((_E * _CHUNK,), jnp.float32),
        ],
    )(functools.partial(_sc_mask_kernel, n_tokens=B))
    p = sc_mask(lt.reshape(E * B)).reshape(E, B).T

    Tb = 512
    Hb = 512
    return pl.pallas_call(
        _combine_kernel,
        grid=(H // Hb, B // Tb),
        in_specs=[
            pl.BlockSpec((Tb, E), lambda h, t: (t, 0)),
            pl.BlockSpec((Tb, D), lambda h, t: (t, 0)),
            pl.BlockSpec((E, D, Hb), lambda h, t: (0, 0, h)),
            pl.BlockSpec((E, Hb), lambda h, t: (0, h)),
        ],
        out_specs=pl.BlockSpec((Tb, Hb), lambda h, t: (t, h)),
        out_shape=jax.ShapeDtypeStruct((B, H), jnp.float32),
        compiler_params=pltpu.CompilerParams(vmem_limit_bytes=67108864),
    )(p, x, We, be)


# speedup vs baseline: 1.0891x; 1.0891x over previous
import jax
import jax.numpy as jnp
from jax.experimental import pallas as pl
from jax.experimental.pallas import tpu as pltpu

_E = 5
_K = 3


def _fused_moe_kernel(x_ref, wg_ref, bg_ref, we_ref, be_ref, o_ref):
    x = x_ref[...]
    logits = jnp.dot(x, wg_ref[...],
                     preferred_element_type=jnp.float32) + bg_ref[...]
    g = jax.nn.softmax(logits, axis=-1)
    cols = [g[:, e:e + 1] for e in range(_E)]
    ps = []
    for e in range(_E):
        cnt = jnp.zeros_like(cols[e])
        for e2 in range(_E):
            if e2 == e:
                continue
            if e2 < e:
                beats = cols[e2] >= cols[e]
            else:
                beats = cols[e2] > cols[e]
            cnt = cnt + beats.astype(jnp.float32)
        ps.append(jnp.where(cnt < float(_K), cols[e], 0.0))
    accs = [None, None]
    for e in range(_E):
        y_e = jnp.dot(x, we_ref[e], preferred_element_type=jnp.float32)
        term = ps[e] * (y_e + be_ref[e][None, :])
        k = e % 2
        accs[k] = term if accs[k] is None else accs[k] + term
    o_ref[...] = accs[0] + accs[1]


def kernel(x, Wg, bg, We, be):
    B, D = x.shape
    E, _, H = We.shape
    Tb = 512
    Hb = 512
    bg2 = bg.reshape(1, E)
    grid = (H // Hb, B // Tb)
    return pl.pallas_call(
        _fused_moe_kernel,
        grid=grid,
        in_specs=[
            pl.BlockSpec((Tb, D), lambda h, t: (t, 0)),
            pl.BlockSpec((D, E), lambda h, t: (0, 0)),
            pl.BlockSpec((1, E), lambda h, t: (0, 0)),
            pl.BlockSpec((E, D, Hb), lambda h, t: (0, 0, h)),
            pl.BlockSpec((E, Hb), lambda h, t: (0, h)),
        ],
        out_specs=pl.BlockSpec((Tb, Hb), lambda h, t: (t, h)),
        out_shape=jax.ShapeDtypeStruct((B, H), jnp.float32),
        compiler_params=pltpu.CompilerParams(
            vmem_limit_bytes=67108864,
            dimension_semantics=("parallel", "parallel"),
        ),
    )(x, Wg, bg2, We, be)


# speedup vs baseline: 1.0923x; 1.0029x over previous
import jax
import jax.numpy as jnp
from jax.experimental import pallas as pl
from jax.experimental.pallas import tpu as pltpu

_E = 5
_K = 3


def _fused_moe_kernel(x_ref, wg_ref, bg_ref, we_ref, be_ref, o_ref):
    x = x_ref[...]
    logits = jnp.dot(x, wg_ref[...],
                     preferred_element_type=jnp.float32) + bg_ref[...]
    g = jax.nn.softmax(logits, axis=-1)
    cols = [g[:, e:e + 1] for e in range(_E)]
    ps = []
    for e in range(_E):
        cnt = jnp.zeros_like(cols[e])
        for e2 in range(_E):
            if e2 == e:
                continue
            if e2 < e:
                beats = cols[e2] >= cols[e]
            else:
                beats = cols[e2] > cols[e]
            cnt = cnt + beats.astype(jnp.float32)
        ps.append(jnp.where(cnt < float(_K), cols[e], 0.0))
    acc = jnp.zeros((x.shape[0], o_ref.shape[1]), jnp.float32)
    for e in range(_E):
        y_e = jnp.dot(x, we_ref[e], preferred_element_type=jnp.float32)
        acc = acc + ps[e] * (y_e + be_ref[e][None, :])
    o_ref[...] = acc


def kernel(x, Wg, bg, We, be):
    B, D = x.shape
    E, _, H = We.shape
    Tb = 512
    Hb = 512
    bg2 = bg.reshape(1, E)
    grid = (H // Hb, B // Tb)
    return pl.pallas_call(
        _fused_moe_kernel,
        grid=grid,
        in_specs=[
            pl.BlockSpec((Tb, D), lambda h, t: (t, 0)),
            pl.BlockSpec((D, E), lambda h, t: (0, 0)),
            pl.BlockSpec((1, E), lambda h, t: (0, 0)),
            pl.BlockSpec((E, D, Hb), lambda h, t: (0, 0, h)),
            pl.BlockSpec((E, Hb), lambda h, t: (0, h)),
        ],
        out_specs=pl.BlockSpec((Tb, Hb), lambda h, t: (t, h)),
        out_shape=jax.ShapeDtypeStruct((B, H), jnp.float32),
        compiler_params=pltpu.CompilerParams(
            vmem_limit_bytes=67108864,
            dimension_semantics=("parallel", "parallel"),
        ),
    )(x, Wg, bg2, We, be)
